# baseline (speedup 1.0000x reference)
import jax
import jax.numpy as jnp
from jax.experimental import pallas as pl

B = 16384
D = 128
BS = 2048

_DN_WX = (((1,), (0,)), ((), ()))
_DN_FLIP = (((0,), (1,)), ((), ()))


def _fused_mlp(xt_ref, w0a_ref, b0a_ref, w0b_ref, b0b_ref, w1p_ref, b1p_ref,
               w1q_ref, b1q_ref, o_ref):
    f32 = jnp.float32
    bf16 = jnp.bfloat16
    x0 = xt_ref[:D, :].astype(bf16)
    x1 = xt_ref[D:, :].astype(bf16)
    w0a = w0a_ref[0].astype(bf16)
    w0b = w0b_ref[0].astype(bf16)
    w1p = w1p_ref[0].astype(bf16)
    w1q = w1q_ref[0].astype(bf16)
    ba = b0a_ref[0][:, None]
    bb = b0b_ref[0][:, None]
    bp = b1p_ref[0][:, None]
    bq = b1q_ref[0:1, :]
    h0 = jnp.maximum(
        jax.lax.dot_general(w0a, x0, _DN_WX, preferred_element_type=f32) + ba,
        0.0).astype(bf16)
    x = jnp.maximum(
        jax.lax.dot_general(w0b, h0, _DN_WX, preferred_element_type=f32) + bb,
        0.0).astype(bf16)
    h1 = jnp.maximum(
        jax.lax.dot_general(w1p, x1, _DN_WX, preferred_element_type=f32) + bp,
        0.0).astype(bf16)
    o_ref[...] = (
        jax.lax.dot_general(x, w1q[:, :D], _DN_FLIP, preferred_element_type=f32)
        + jax.lax.dot_general(h1, w1q[:, D:], _DN_FLIP, preferred_element_type=f32)
        + bq)


def kernel(input_val, W0a, b0a, W0b, b0b, W1p, b1p, W1q, b1q):
    xt = input_val.T
    n_blocks = B // BS
    wspec = pl.BlockSpec((1, D, D), lambda i: (0, 0, 0))
    wspec2 = pl.BlockSpec((1, D, 2 * D), lambda i: (0, 0, 0))
    bspec = pl.BlockSpec((8, D), lambda i: (0, 0))
    out = pl.pallas_call(
        _fused_mlp,
        grid=(n_blocks,),
        in_specs=[
            pl.BlockSpec((2 * D, BS), lambda i: (0, i)),
            wspec, bspec, wspec, bspec, wspec, bspec, wspec2, bspec,
        ],
        out_specs=pl.BlockSpec((BS, D), lambda i: (i, 0)),
        out_shape=jax.ShapeDtypeStruct((B, D), jnp.float32),
    )(xt, W0a, b0a, W0b, b0b, W1p, b1p, W1q, b1q)
    return out


# speedup vs baseline: 1.2184x; 1.2184x over previous
import jax
import jax.numpy as jnp
from jax.experimental import pallas as pl

B = 16384
D = 128
BS = 8192

_DN_WX = (((1,), (0,)), ((), ()))
_DN_FLIP = (((0,), (1,)), ((), ()))


def _fused_mlp(xt_ref, w0a_ref, b0a_ref, w0b_ref, b0b_ref, w1p_ref, b1p_ref,
               w1q_ref, b1q_ref, o_ref):
    f32 = jnp.float32
    bf16 = jnp.bfloat16
    x0 = xt_ref[:D, :].astype(bf16)
    x1 = xt_ref[D:, :].astype(bf16)
    w0a = w0a_ref[0].astype(bf16)
    w0b = w0b_ref[0].astype(bf16)
    w1p = w1p_ref[0].astype(bf16)
    w1q = w1q_ref[0].astype(bf16)
    ba = b0a_ref[0][:, None]
    bb = b0b_ref[0][:, None]
    bp = b1p_ref[0][:, None]
    bq = b1q_ref[0:1, :]
    h0 = jnp.maximum(
        jax.lax.dot_general(w0a, x0, _DN_WX, preferred_element_type=f32) + ba,
        0.0).astype(bf16)
    x = jnp.maximum(
        jax.lax.dot_general(w0b, h0, _DN_WX, preferred_element_type=f32) + bb,
        0.0).astype(bf16)
    h1 = jnp.maximum(
        jax.lax.dot_general(w1p, x1, _DN_WX, preferred_element_type=f32) + bp,
        0.0).astype(bf16)
    o_ref[...] = (
        jax.lax.dot_general(x, w1q[:, :D], _DN_FLIP, preferred_element_type=f32)
        + jax.lax.dot_general(h1, w1q[:, D:], _DN_FLIP, preferred_element_type=f32)
        + bq)


def kernel(input_val, W0a, b0a, W0b, b0b, W1p, b1p, W1q, b1q):
    xt = input_val.T
    n_blocks = B // BS
    wspec = pl.BlockSpec((1, D, D), lambda i: (0, 0, 0))
    wspec2 = pl.BlockSpec((1, D, 2 * D), lambda i: (0, 0, 0))
    bspec = pl.BlockSpec((8, D), lambda i: (0, 0))
    out = pl.pallas_call(
        _fused_mlp,
        grid=(n_blocks,),
        in_specs=[
            pl.BlockSpec((2 * D, BS), lambda i: (0, i)),
            wspec, bspec, wspec, bspec, wspec, bspec, wspec2, bspec,
        ],
        out_specs=pl.BlockSpec((BS, D), lambda i: (i, 0)),
        out_shape=jax.ShapeDtypeStruct((B, D), jnp.float32),
    )(xt, W0a, b0a, W0b, b0b, W1p, b1p, W1q, b1q)
    return out
